# baseline (speedup 1.0000x reference)
import functools

import jax
import jax.numpy as jnp
from jax import lax
from jax.experimental import pallas as pl
from jax.experimental.pallas import tpu as pltpu
from jax.experimental.pallas import tpu_sc as plsc

_L = 16


def _make_sc_spmm(N, B, K):
    mesh = plsc.VectorSubcoreMesh(core_axis_name="c", subcore_axis_name="s")
    NW = 32
    R = N // NW

    @functools.partial(
        pl.kernel,
        mesh=mesh,
        out_type=jax.ShapeDtypeStruct((N * B,), jnp.float32),
        compiler_params=pltpu.CompilerParams(needs_layout_passes=False),
        scratch_types=[
            pltpu.VMEM((16,), jnp.int32),
            pltpu.VMEM((K,), jnp.int32),
            pltpu.VMEM((K,), jnp.int32),
            pltpu.VMEM((K,), jnp.int32),
            pltpu.VMEM((K,), jnp.int32),
            pltpu.VMEM((K,), jnp.float32),
            pltpu.VMEM((K,), jnp.float32),
            pltpu.VMEM((K, B), jnp.float32),
            pltpu.VMEM((K, B), jnp.float32),
            pltpu.VMEM((R * B,), jnp.float32),
            pltpu.SemaphoreType.DMA,
            pltpu.SemaphoreType.DMA,
            pltpu.SemaphoreType.DMA,
            pltpu.SemaphoreType.DMA,
            pltpu.SemaphoreType.DMA,
            pltpu.SemaphoreType.DMA,
        ],
    )
    def sc_spmm(xT_hbm, col_hbm, adr_hbm, val_hbm, offs_hbm, out_hbm,
                ov, colv0, colv1, adv0, adv1, vav0, vav1,
                rows0, rows1, acc, ssem0, ssem1, csem0, csem1, gsem0, gsem1):
        c = lax.axis_index("c")
        s = lax.axis_index("s")
        wid = s * 2 + c
        r0 = wid * R

        pltpu.sync_copy(offs_hbm.at[pl.ds(r0, 16)], ov)
        s0 = jnp.min(ov[pl.ds(0, _L)])
        pltpu.sync_copy(offs_hbm.at[pl.ds(r0 + R, 16)], ov)
        e0 = jnp.min(ov[pl.ds(0, _L)])
        a0 = (s0 >> 3) << 3
        nb = (e0 - a0 + (K - 1)) >> 7

        zero = jnp.zeros((_L,), jnp.float32)

        @plsc.parallel_loop(0, R, 1, unroll=4)
        def _(r):
            for cc in range(B // _L):
                acc[pl.ds(r * B + cc * _L, _L)] = zero

        iota = lax.iota(jnp.int32, _L)
        abase = r0 * B
        accs = [acc.at[pl.ds(cc * _L, R * B - (B - _L))]
                for cc in range(B // _L)]

        def stage_av(t, adv, vav, ssem):
            off = pl.multiple_of(a0 + t * K, 8)
            pltpu.async_copy(adr_hbm.at[pl.ds(off, K)], adv, ssem)
            pltpu.async_copy(val_hbm.at[pl.ds(off, K)], vav, ssem)

        def stage_av_wait(adv, vav, ssem):
            pltpu.make_async_copy(adr_hbm.at[pl.ds(0, K)], adv, ssem).wait()
            pltpu.make_async_copy(val_hbm.at[pl.ds(0, K)], vav, ssem).wait()

        def stage_col(t, colv, csem):
            off = pl.multiple_of(a0 + t * K, 8)
            pltpu.async_copy(col_hbm.at[pl.ds(off, K)], colv, csem)

        def stage_col_wait(colv, csem):
            pltpu.make_async_copy(col_hbm.at[pl.ds(0, K)], colv, csem).wait()

        def gather(colv, rows, gsem):
            pltpu.async_copy(xT_hbm.at[colv], rows, gsem)

        def gather_wait(rows, gsem):
            pltpu.make_async_copy(xT_hbm.at[pl.ds(0, K)], rows, gsem).wait()

        def compute(t, adv, vav, rows):
            off = pl.multiple_of(a0 + t * K, 8)

            @plsc.parallel_loop(0, K, 1, unroll=16)
            def _(j):
                g = off + j
                valid = jnp.logical_and(g >= s0, g < e0)
                w = jnp.where(valid, jnp.float32(1.0), jnp.float32(0.0))
                jm = (j >> 4) << 4
                sel = jnp.full((_L,), j & 15, jnp.int32)
                v16 = vav[pl.ds(jm, _L)]
                a16 = adv[pl.ds(jm, _L)]
                v = v16.at[sel].get(mode="promise_in_bounds") * w
                a = a16.at[sel].get(mode="promise_in_bounds") - abase
                a = jnp.minimum(jnp.maximum(a, 0), (R - 1) * B) + iota
                xs = [rows[j, pl.ds(cc * _L, _L)] for cc in range(B // _L)]
                for cc in range(B // _L):
                    plsc.addupdate_scatter(accs[cc], [a], xs[cc] * v)

        stage_col(0, colv0, csem0)
        stage_col(1, colv1, csem1)
        stage_av(0, adv0, vav0, ssem0)
        stage_av(1, adv1, vav1, ssem1)
        stage_col_wait(colv0, csem0)
        gather(colv0, rows0, gsem0)
        stage_col(2, colv0, csem0)
        stage_col_wait(colv1, csem1)
        gather(colv1, rows1, gsem1)
        stage_col(3, colv1, csem1)
        nh = (nb + 1) >> 1

        def pair(i, carry):
            t0 = 2 * i
            gather_wait(rows0, gsem0)
            stage_av_wait(adv0, vav0, ssem0)
            compute(t0, adv0, vav0, rows0)
            stage_av(t0 + 2, adv0, vav0, ssem0)
            stage_col_wait(colv0, csem0)
            gather(colv0, rows0, gsem0)
            stage_col(t0 + 4, colv0, csem0)
            gather_wait(rows1, gsem1)
            stage_av_wait(adv1, vav1, ssem1)
            compute(t0 + 1, adv1, vav1, rows1)
            stage_av(t0 + 3, adv1, vav1, ssem1)
            stage_col_wait(colv1, csem1)
            gather(colv1, rows1, gsem1)
            stage_col(t0 + 5, colv1, csem1)
            return carry

        lax.fori_loop(0, nh, pair, 0)
        gather_wait(rows0, gsem0)
        gather_wait(rows1, gsem1)
        stage_av_wait(adv0, vav0, ssem0)
        stage_av_wait(adv1, vav1, ssem1)
        stage_col_wait(colv0, csem0)
        stage_col_wait(colv1, csem1)

        pltpu.sync_copy(acc, out_hbm.at[pl.ds(r0 * B, R * B)])

    return sc_spmm


def kernel(x, values, row_ids, col_idx, row_offs):
    B, N = x.shape
    NNZ = values.shape[0]
    K = 128

    xT = x.T
    pad = 7 * K + 8
    colp = jnp.concatenate([col_idx, jnp.zeros((pad,), jnp.int32)])
    adrp = jnp.concatenate([row_ids * B, jnp.zeros((pad,), jnp.int32)])
    valp = jnp.concatenate([values, jnp.zeros((pad,), values.dtype)])
    offsp = jnp.concatenate([row_offs, jnp.full((15,), NNZ, jnp.int32)])

    outT = _make_sc_spmm(N, B, K)(xT, colp, adrp, valp, offsp)
    return outT.reshape(N, B).T


# speedup vs baseline: 1.0144x; 1.0144x over previous
import functools

import jax
import jax.numpy as jnp
from jax import lax
from jax.experimental import pallas as pl
from jax.experimental.pallas import tpu as pltpu
from jax.experimental.pallas import tpu_sc as plsc

_L = 16


def _make_sc_spmm(N, B, K):
    mesh = plsc.VectorSubcoreMesh(core_axis_name="c", subcore_axis_name="s")
    NW = 32
    R = N // NW

    @functools.partial(
        pl.kernel,
        mesh=mesh,
        out_type=jax.ShapeDtypeStruct((N * B,), jnp.float32),
        compiler_params=pltpu.CompilerParams(needs_layout_passes=False),
        scratch_types=[
            pltpu.VMEM((16,), jnp.int32),
            pltpu.VMEM((K,), jnp.int32),
            pltpu.VMEM((K,), jnp.int32),
            pltpu.VMEM((K,), jnp.int32),
            pltpu.VMEM((K,), jnp.int32),
            pltpu.VMEM((K,), jnp.float32),
            pltpu.VMEM((K,), jnp.float32),
            pltpu.VMEM((K, B), jnp.float32),
            pltpu.VMEM((K, B), jnp.float32),
            pltpu.VMEM((R * B,), jnp.float32),
            pltpu.SemaphoreType.DMA,
            pltpu.SemaphoreType.DMA,
            pltpu.SemaphoreType.DMA,
            pltpu.SemaphoreType.DMA,
            pltpu.SemaphoreType.DMA,
            pltpu.SemaphoreType.DMA,
        ],
    )
    def sc_spmm(xT_hbm, col_hbm, adr_hbm, val_hbm, offs_hbm, out_hbm,
                ov, colv0, colv1, adv0, adv1, vav0, vav1,
                rows0, rows1, acc, ssem0, ssem1, csem0, csem1, gsem0, gsem1):
        c = lax.axis_index("c")
        s = lax.axis_index("s")
        wid = s * 2 + c
        r0 = wid * R

        pltpu.sync_copy(offs_hbm.at[pl.ds(r0, 16)], ov)
        s0 = jnp.min(ov[pl.ds(0, _L)])
        pltpu.sync_copy(offs_hbm.at[pl.ds(r0 + R, 16)], ov)
        e0 = jnp.min(ov[pl.ds(0, _L)])
        a0 = (s0 >> 3) << 3
        nb = (e0 - a0 + (K - 1)) >> 7

        zero = jnp.zeros((_L,), jnp.float32)

        @plsc.parallel_loop(0, R, 1, unroll=4)
        def _(r):
            for cc in range(B // _L):
                acc[pl.ds(r * B + cc * _L, _L)] = zero

        iota = lax.iota(jnp.int32, _L)
        abase = r0 * B
        accs = [acc.at[pl.ds(cc * _L, R * B - (B - _L))]
                for cc in range(B // _L)]

        def stage_av(t, adv, vav, ssem):
            off = pl.multiple_of(a0 + t * K, 8)
            pltpu.async_copy(adr_hbm.at[pl.ds(off, K)], adv, ssem)
            pltpu.async_copy(val_hbm.at[pl.ds(off, K)], vav, ssem)

        def stage_av_wait(adv, vav, ssem):
            pltpu.make_async_copy(adr_hbm.at[pl.ds(0, K)], adv, ssem).wait()
            pltpu.make_async_copy(val_hbm.at[pl.ds(0, K)], vav, ssem).wait()

        def stage_col(t, colv, csem):
            off = pl.multiple_of(a0 + t * K, 8)
            pltpu.async_copy(col_hbm.at[pl.ds(off, K)], colv, csem)

        def stage_col_wait(colv, csem):
            pltpu.make_async_copy(col_hbm.at[pl.ds(0, K)], colv, csem).wait()

        def gather(colv, rows, gsem):
            pltpu.async_copy(xT_hbm.at[colv], rows, gsem)

        def gather_wait(rows, gsem):
            pltpu.make_async_copy(xT_hbm.at[pl.ds(0, K)], rows, gsem).wait()

        def compute(t, adv, vav, rows):
            off = pl.multiple_of(a0 + t * K, 8)

            @plsc.parallel_loop(0, K, 1, unroll=8)
            def _(j):
                g = off + j
                valid = jnp.logical_and(g >= s0, g < e0)
                w = jnp.where(valid, jnp.float32(1.0), jnp.float32(0.0))
                jm = (j >> 4) << 4
                sel = jnp.full((_L,), j & 15, jnp.int32)
                v16 = vav[pl.ds(jm, _L)]
                a16 = adv[pl.ds(jm, _L)]
                v = v16.at[sel].get(mode="promise_in_bounds") * w
                a = a16.at[sel].get(mode="promise_in_bounds") - abase
                a = jnp.minimum(jnp.maximum(a, 0), (R - 1) * B) + iota
                xs = [rows[j, pl.ds(cc * _L, _L)] for cc in range(B // _L)]
                for cc in range(B // _L):
                    plsc.addupdate_scatter(accs[cc], [a], xs[cc] * v)

        stage_col(0, colv0, csem0)
        stage_col(1, colv1, csem1)
        stage_av(0, adv0, vav0, ssem0)
        stage_av(1, adv1, vav1, ssem1)
        stage_col_wait(colv0, csem0)
        gather(colv0, rows0, gsem0)
        stage_col(2, colv0, csem0)
        stage_col_wait(colv1, csem1)
        gather(colv1, rows1, gsem1)
        stage_col(3, colv1, csem1)
        nh = (nb + 1) >> 1

        def pair(i, carry):
            t0 = 2 * i
            gather_wait(rows0, gsem0)
            stage_av_wait(adv0, vav0, ssem0)
            compute(t0, adv0, vav0, rows0)
            stage_av(t0 + 2, adv0, vav0, ssem0)
            stage_col_wait(colv0, csem0)
            gather(colv0, rows0, gsem0)
            stage_col(t0 + 4, colv0, csem0)
            gather_wait(rows1, gsem1)
            stage_av_wait(adv1, vav1, ssem1)
            compute(t0 + 1, adv1, vav1, rows1)
            stage_av(t0 + 3, adv1, vav1, ssem1)
            stage_col_wait(colv1, csem1)
            gather(colv1, rows1, gsem1)
            stage_col(t0 + 5, colv1, csem1)
            return carry

        lax.fori_loop(0, nh, pair, 0)
        gather_wait(rows0, gsem0)
        gather_wait(rows1, gsem1)
        stage_av_wait(adv0, vav0, ssem0)
        stage_av_wait(adv1, vav1, ssem1)
        stage_col_wait(colv0, csem0)
        stage_col_wait(colv1, csem1)

        pltpu.sync_copy(acc, out_hbm.at[pl.ds(r0 * B, R * B)])

    return sc_spmm


def kernel(x, values, row_ids, col_idx, row_offs):
    B, N = x.shape
    NNZ = values.shape[0]
    K = 128

    xT = x.T
    pad = 7 * K + 8
    colp = jnp.concatenate([col_idx, jnp.zeros((pad,), jnp.int32)])
    adrp = jnp.concatenate([row_ids * B, jnp.zeros((pad,), jnp.int32)])
    valp = jnp.concatenate([values, jnp.zeros((pad,), values.dtype)])
    offsp = jnp.concatenate([row_offs, jnp.full((15,), NNZ, jnp.int32)])

    outT = _make_sc_spmm(N, B, K)(xT, colp, adrp, valp, offsp)
    return outT.reshape(N, B).T
